# baseline (speedup 1.0000x reference)
import jax
import jax.numpy as jnp
from jax.experimental import pallas as pl
from jax.experimental.pallas import tpu as pltpu

RATE = 0.8
BATCH = 16384
NCLS = 1000
BLOCK_ROWS = 1024
NBLOCKS = BATCH // BLOCK_ROWS
HALF = NBLOCKS // 2
KEEP = int(BATCH * RATE)
_INTERPRET = False


def _ce_block(block, tgt):
    s = jnp.sum(jnp.exp(jnp.minimum(block, 60.0)), axis=1)
    lse = jnp.log(s)
    col = jax.lax.broadcasted_iota(jnp.int32, block.shape, 1)
    tsel = jnp.sum(jnp.where(col == tgt[:, None], block, 0.0), axis=1)
    return lse - tsel


def _ohem_kernel(pred_a, pred_b, tgt_ref, out_ref, loss_scratch):
    i = pl.program_id(0)
    loss_scratch[i, :] = _ce_block(
        pred_a[...], tgt_ref[pl.ds(i * BLOCK_ROWS, BLOCK_ROWS)]
    )
    loss_scratch[i + HALF, :] = _ce_block(
        pred_b[...], tgt_ref[pl.ds((i + HALF) * BLOCK_ROWS, BLOCK_ROWS)]
    )

    @pl.when(i == HALF - 1)
    def _select():
        v = loss_scratch[...]
        bits = jax.lax.bitcast_convert_type(v, jnp.int32)

        def body(_, lohi):
            lo, hi = lohi
            mid = lo + (hi - lo + 1) // 2
            cnt = jnp.sum((bits >= mid).astype(jnp.int32))
            take = cnt >= KEEP
            return jnp.where(take, mid, lo), jnp.where(take, hi, mid - 1)

        lo, _ = jax.lax.fori_loop(
            0, 31, body, (jnp.int32(0), jnp.int32(0x7F7FFFFF))
        )
        tval = jax.lax.bitcast_convert_type(lo, jnp.float32)
        gt = bits > lo
        cnt_gt = jnp.sum(gt.astype(jnp.int32))
        sum_gt = jnp.sum(jnp.where(gt, v, 0.0))
        total = sum_gt + (KEEP - cnt_gt).astype(jnp.float32) * tval
        out_ref[...] = (total / KEEP).reshape(1, 1)


@jax.jit
def _ohem(cls_pred, cls_target):
    out = pl.pallas_call(
        _ohem_kernel,
        grid=(HALF,),
        in_specs=[
            pl.BlockSpec((BLOCK_ROWS, NCLS), lambda i: (i, 0)),
            pl.BlockSpec((BLOCK_ROWS, NCLS), lambda i: (i + HALF, 0)),
            pl.BlockSpec((BATCH,), lambda i: (0,)),
        ],
        out_specs=pl.BlockSpec((1, 1), lambda i: (0, 0)),
        out_shape=jax.ShapeDtypeStruct((1, 1), jnp.float32),
        scratch_shapes=[pltpu.VMEM((NBLOCKS, BLOCK_ROWS), jnp.float32)],
        interpret=_INTERPRET,
    )(cls_pred, cls_pred, cls_target)
    return out[0, 0]


def kernel(cls_pred, cls_target):
    return _ohem(cls_pred, cls_target.astype(jnp.int32))
